# initial kernel scaffold (unmeasured)
import jax
import jax.numpy as jnp
from jax import lax
from jax.experimental import pallas as pl
from jax.experimental.pallas import tpu as pltpu

N_DEV = 16
N_GLOBAL = 16384
EPS = 1e-5
R, C = 16, 128


def kernel(x, gamma):
    m, n_per = x.shape

    def body(x_ref, g_ref, out_ref, comm_ref, send_sems, recv_sems):
        my = lax.axis_index("i")

        x3 = x_ref[...].reshape(R, C, n_per)
        comm_ref[0, :, :] = jnp.sum(x3 * x3, axis=2)

        rdmas = []
        for o in range(1, N_DEV):
            dst = lax.rem(my + o, N_DEV)
            rdma = pltpu.make_async_remote_copy(
                src_ref=comm_ref.at[0],
                dst_ref=comm_ref.at[o],
                send_sem=send_sems.at[o],
                recv_sem=recv_sems.at[o],
                device_id=(dst,),
                device_id_type=pl.DeviceIdType.MESH,
            )
            rdma.start()
            rdmas.append(rdma)

        xg = x_ref[...] * g_ref[...]

        for rdma in rdmas:
            rdma.wait()

        total = jnp.sum(comm_ref[...], axis=0)
        inv = lax.rsqrt(total / N_GLOBAL + EPS)
        out_ref[...] = (xg.reshape(R, C, n_per) * inv[:, :, None]).reshape(
            m, n_per
        )

    return pl.pallas_call(
        body,
        out_shape=jax.ShapeDtypeStruct((m, n_per), jnp.float32),
        in_specs=[
            pl.BlockSpec(memory_space=pltpu.VMEM),
            pl.BlockSpec(memory_space=pltpu.VMEM),
        ],
        out_specs=pl.BlockSpec(memory_space=pltpu.VMEM),
        scratch_shapes=[
            pltpu.VMEM((N_DEV, R, C), jnp.float32),
            pltpu.SemaphoreType.DMA((N_DEV,)),
            pltpu.SemaphoreType.DMA((N_DEV,)),
        ],
        compiler_params=pltpu.CompilerParams(collective_id=0),
    )(x, gamma.reshape(1, n_per))


# baseline (device time: 22806 ns/iter reference)
import jax
import jax.numpy as jnp
from jax import lax
from jax.experimental import pallas as pl
from jax.experimental.pallas import tpu as pltpu

N_DEV = 16
N_GLOBAL = 16384
EPS = 1e-5
R, C = 16, 128


def kernel(x, gamma):
    m, n_per = x.shape

    def body(x_ref, g_ref, out_ref, comm_ref, send_sems, recv_sems):
        my = lax.axis_index("i")

        x3 = x_ref[...].reshape(R, C, n_per)
        comm_ref[0, :, :] = jnp.sum(x3 * x3, axis=2)

        rdmas = []
        for o in range(1, N_DEV):
            dst = lax.rem(my + o, N_DEV)
            rdma = pltpu.make_async_remote_copy(
                src_ref=comm_ref.at[0],
                dst_ref=comm_ref.at[o],
                send_sem=send_sems.at[o],
                recv_sem=recv_sems.at[o],
                device_id=(dst,),
                device_id_type=pl.DeviceIdType.MESH,
            )
            rdma.start()
            rdmas.append(rdma)

        xg = x_ref[...] * g_ref[...]

        for rdma in rdmas:
            rdma.wait()

        total = jnp.sum(comm_ref[...], axis=0)
        inv = lax.rsqrt(total / N_GLOBAL + EPS)
        out_ref[...] = (xg.reshape(R, C, n_per) * inv[:, :, None]).reshape(
            m, n_per
        )

    return pl.pallas_call(
        body,
        out_shape=jax.ShapeDtypeStruct((m, n_per), jnp.float32),
        in_specs=[
            pl.BlockSpec(memory_space=pltpu.VMEM),
            pl.BlockSpec(memory_space=pltpu.VMEM),
        ],
        out_specs=pl.BlockSpec(memory_space=pltpu.VMEM),
        scratch_shapes=[
            pltpu.VMEM((N_DEV, R, C), jnp.float32),
            pltpu.SemaphoreType.DMA((N_DEV,)),
            pltpu.SemaphoreType.DMA((N_DEV,)),
        ],
    )(x, gamma.reshape(1, n_per))


# device time: 16904 ns/iter; 1.3491x vs baseline; 1.3491x over previous
import jax
import jax.numpy as jnp
from jax import lax
from jax.experimental import pallas as pl
from jax.experimental.pallas import tpu as pltpu

N_DEV = 16
N_GLOBAL = 16384
EPS = 1e-5
R, C = 16, 128


def kernel(x, gamma):
    m, n_per = x.shape

    def body(x_ref, g_ref, out_ref, comm_ref, send_sems, recv_sems):
        my = lax.axis_index("i")

        barrier_sem = pltpu.get_barrier_semaphore()
        for o in range(1, N_DEV):
            pl.semaphore_signal(
                barrier_sem,
                inc=1,
                device_id=(lax.rem(my + o, N_DEV),),
                device_id_type=pl.DeviceIdType.MESH,
            )

        x3 = x_ref[...].reshape(R, C, n_per)
        comm_ref[0, :, :] = jnp.sum(x3 * x3, axis=2)

        pl.semaphore_wait(barrier_sem, N_DEV - 1)

        rdmas = []
        for o in range(1, N_DEV):
            dst = lax.rem(my + o, N_DEV)
            rdma = pltpu.make_async_remote_copy(
                src_ref=comm_ref.at[0],
                dst_ref=comm_ref.at[o],
                send_sem=send_sems.at[o],
                recv_sem=recv_sems.at[o],
                device_id=(dst,),
                device_id_type=pl.DeviceIdType.MESH,
            )
            rdma.start()
            rdmas.append(rdma)

        xg = x_ref[...] * g_ref[...]

        for rdma in rdmas:
            rdma.wait_recv()

        total = jnp.sum(comm_ref[...], axis=0)
        inv = lax.rsqrt(total / N_GLOBAL + EPS)
        out_ref[...] = (xg.reshape(R, C, n_per) * inv[:, :, None]).reshape(
            m, n_per
        )

        for rdma in rdmas:
            rdma.wait_send()

    return pl.pallas_call(
        body,
        out_shape=jax.ShapeDtypeStruct((m, n_per), jnp.float32),
        in_specs=[
            pl.BlockSpec(memory_space=pltpu.VMEM),
            pl.BlockSpec(memory_space=pltpu.VMEM),
        ],
        out_specs=pl.BlockSpec(memory_space=pltpu.VMEM),
        scratch_shapes=[
            pltpu.VMEM((N_DEV, R, C), jnp.float32),
            pltpu.SemaphoreType.DMA((N_DEV,)),
            pltpu.SemaphoreType.DMA((N_DEV,)),
        ],
        compiler_params=pltpu.CompilerParams(collective_id=0),
    )(x, gamma.reshape(1, n_per))
